# Initial kernel scaffold; baseline (speedup 1.0000x reference)
import jax
import jax.numpy as jnp
from jax.experimental import pallas as pl


def kernel(x, Wg, W1, b1, W2, b2):
    raise NotImplementedError("write your pallas kernel here")


# speedup vs baseline: 1.6840x; 1.6840x over previous
import functools

import jax
import jax.numpy as jnp
from jax.experimental import pallas as pl
from jax.experimental.pallas import tpu as pltpu

_INTERPRET = False

E = 8
D = 768
H = 768
T = 4096
BTR = 512
BT = 128
NT = T // BT
NW = NT + E


def _router_body(x_ref, wg_ref, eid_ref, counts_ref, loss_ref, acc_ref):
    i = pl.program_id(0)
    n = pl.num_programs(0)
    logits = jnp.dot(x_ref[...], wg_ref[...], preferred_element_type=jnp.float32)
    lane = jax.lax.broadcasted_iota(jnp.int32, logits.shape, 1)
    logits = jnp.where(lane < E, logits, -jnp.inf)
    m = jnp.max(logits, axis=1, keepdims=True)
    eid = jnp.min(jnp.where(logits == m, lane, jnp.int32(2**30)), axis=1,
                  keepdims=True)
    eid_ref[...] = eid
    onehot = jnp.where((lane == eid) & (lane < E), jnp.float32(1.0),
                       jnp.float32(0.0))

    @pl.when(i == 0)
    def _():
        acc_ref[...] = jnp.zeros_like(acc_ref)

    acc_ref[...] += jnp.sum(onehot, axis=0, keepdims=True)

    @pl.when(i == n - 1)
    def _():
        c = acc_ref[...]
        counts_ref[...] = c
        lane1 = lane[:1, :]
        mean = jnp.sum(jnp.where(lane1 < E, c, 0.0)) / jnp.float32(E)
        dev = jnp.where(lane1 < E, c - mean, 0.0)
        var = jnp.sum(dev * dev) / jnp.float32(E - 1)
        loss_ref[...] = jnp.full((1, 1), var / (mean * mean + 1e-10),
                                 jnp.float32)


def _router(x, wg_pad):
    return pl.pallas_call(
        _router_body,
        grid=(T // BTR,),
        in_specs=[
            pl.BlockSpec((BTR, D), lambda i: (i, 0)),
            pl.BlockSpec((D, 128), lambda i: (0, 0)),
        ],
        out_specs=[
            pl.BlockSpec((BTR, 1), lambda i: (i, 0)),
            pl.BlockSpec((1, 128), lambda i: (0, 0)),
            pl.BlockSpec((1, 1), lambda i: (0, 0)),
        ],
        out_shape=[
            jax.ShapeDtypeStruct((T, 1), jnp.int32),
            jax.ShapeDtypeStruct((1, 128), jnp.float32),
            jax.ShapeDtypeStruct((1, 1), jnp.float32),
        ],
        scratch_shapes=[pltpu.VMEM((1, 128), jnp.float32)],
        interpret=_INTERPRET,
    )(x, wg_pad)


def _gmm_body(tile_ref, eidw_ref, offs_ref,
              x_ref, w1_ref, b1_ref, w2_ref, b2_ref, y_ref):
    w = pl.program_id(0)
    t = tile_ref[w]
    e = eidw_ref[w]
    s = offs_ref[e]
    epos = offs_ref[e + 1]
    rows = t * BT + jax.lax.broadcasted_iota(jnp.int32, (BT, 1), 0)
    mask = (rows >= s) & (rows < epos)
    h = jnp.maximum(
        jnp.dot(x_ref[...], w1_ref[0], preferred_element_type=jnp.float32)
        + b1_ref[0], 0.0)
    y = jnp.dot(h, w2_ref[0], preferred_element_type=jnp.float32) + b2_ref[0]
    y_ref[...] = jnp.where(mask, y, y_ref[...])


def _gmm(wu_tile, wu_eid, offs, x_sorted, W1, b1, W2, b2):
    grid_spec = pltpu.PrefetchScalarGridSpec(
        num_scalar_prefetch=3,
        grid=(NW,),
        in_specs=[
            pl.BlockSpec((BT, D), lambda w, tr, er, ofr: (tr[w], 0)),
            pl.BlockSpec((1, D, H), lambda w, tr, er, ofr: (er[w], 0, 0)),
            pl.BlockSpec((1, 1, H), lambda w, tr, er, ofr: (er[w], 0, 0)),
            pl.BlockSpec((1, H, D), lambda w, tr, er, ofr: (er[w], 0, 0)),
            pl.BlockSpec((1, 1, D), lambda w, tr, er, ofr: (er[w], 0, 0)),
        ],
        out_specs=pl.BlockSpec((BT, D), lambda w, tr, er, ofr: (tr[w], 0)),
    )
    return pl.pallas_call(
        _gmm_body,
        grid_spec=grid_spec,
        out_shape=jax.ShapeDtypeStruct((T, D), jnp.float32),
        compiler_params=pltpu.CompilerParams(
            dimension_semantics=("arbitrary",)),
        interpret=_INTERPRET,
    )(wu_tile, wu_eid, offs, x_sorted, W1,
      b1.reshape(E, 1, H), W2, b2.reshape(E, 1, D))


def _plan_work_units(offs9):
    interior = offs9[1:E]
    tstart = jnp.arange(NT, dtype=jnp.int32) * BT
    e_start = jnp.sum(interior[None, :] <= tstart[:, None], axis=1)
    e_end = jnp.sum(interior[None, :] <= (tstart + BT - 1)[:, None], axis=1)
    nw = e_end - e_start + 1
    starts = jnp.concatenate(
        [jnp.zeros((1,), jnp.int32), jnp.cumsum(nw)]).astype(jnp.int32)
    n = starts[NT]
    w = jnp.arange(NW, dtype=jnp.int32)
    wc = jnp.minimum(w, n - 1)
    t = jnp.sum(starts[None, :NT] <= wc[:, None], axis=1).astype(jnp.int32) - 1
    eid = e_start[t] + (wc - starts[t])
    return t.astype(jnp.int32), eid.astype(jnp.int32)


def kernel(x, Wg, W1, b1, W2, b2):
    wg_pad = jnp.zeros((D, 128), jnp.float32).at[:, :E].set(Wg)
    eid2d, counts128, loss11 = _router(x, wg_pad)
    eid = eid2d[:, 0]
    counts = counts128[0, :E].astype(jnp.int32)
    loss = loss11[0, 0]
    offs9 = jnp.concatenate(
        [jnp.zeros((1,), jnp.int32), jnp.cumsum(counts)]).astype(jnp.int32)
    offs16 = jnp.zeros((16,), jnp.int32).at[:E + 1].set(offs9)

    sort_idx = jnp.argsort(eid, stable=True)
    x_sorted = jnp.take(x, sort_idx, axis=0)

    wu_tile, wu_eid = _plan_work_units(offs9)
    y_sorted = _gmm(wu_tile, wu_eid, offs16, x_sorted, W1, b1, W2, b2)

    out = jnp.zeros((T, D), jnp.float32).at[sort_idx].set(y_sorted)
    return out, loss, loss
